# baseline (speedup 1.0000x reference)
import functools

import jax
import jax.numpy as jnp
from jax import lax
from jax.experimental import pallas as pl
from jax.experimental.pallas import tpu as pltpu
from jax.experimental.pallas import tpu_sc as plsc

_B, _C, _T = 16384, 14, 200
_NB = 8
_RING = 3


def kernel(chromatogram_batch, perm):
    x = chromatogram_batch
    p = perm.astype(jnp.int32)
    cmap = jnp.concatenate([
        p,
        jnp.array([6], jnp.int32),
        p + 7,
        jnp.array([13], jnp.int32),
        jnp.array([0, 0], jnp.int32),
    ])

    info = plsc.get_sparse_core_info()
    nc, ns = info.num_cores, info.num_subcores
    nw = nc * ns
    bw = _B // nw
    nchunks = bw // _NB
    nrev = (nchunks - _RING) // _RING
    ntail = nchunks - _RING - nrev * _RING
    mesh = plsc.VectorSubcoreMesh(core_axis_name="c", subcore_axis_name="s")

    @functools.partial(
        pl.kernel,
        mesh=mesh,
        out_type=jax.ShapeDtypeStruct((_B, _C, _T), jnp.float32),
        compiler_params=pltpu.CompilerParams(needs_layout_passes=False),
        scratch_types=(
            [pltpu.VMEM((16,), jnp.int32)]
            + [pltpu.VMEM((_NB, _C, _T), jnp.float32) for _ in range(_RING)]
            + [pltpu.SemaphoreType.DMA for _ in range(2 * _RING)]
        ),
    )
    def k(x_hbm, cmap_hbm, out_hbm, cmap_v, *bufs_and_sems):
        bufs = bufs_and_sems[:_RING]
        gsems = bufs_and_sems[_RING:2 * _RING]
        wsems = bufs_and_sems[2 * _RING:]
        wid = lax.axis_index("s") * nc + lax.axis_index("c")
        b0 = wid * bw
        pltpu.sync_copy(cmap_hbm, cmap_v)
        cmapv = cmap_v[...]
        lane = lax.broadcasted_iota(jnp.int32, (16,), 0)
        srcs = [
            jnp.sum(jnp.where(lane == c, cmapv, 0), axis=0) for c in range(_C)
        ]

        def gathers(base, r):
            return [
                pltpu.async_copy(
                    x_hbm.at[pl.ds(base, _NB), pl.ds(srcs[c], 1)],
                    bufs[r].at[pl.ds(0, _NB), pl.ds(c, 1)],
                    gsems[r],
                )
                for c in range(_C)
            ]

        def write(base, r):
            pltpu.async_copy(bufs[r], out_hbm.at[pl.ds(base, _NB)], wsems[r])

        def drain_write(r):
            pltpu.make_async_copy(
                x_hbm.at[pl.ds(0, _NB)], bufs[r], wsems[r]
            ).wait()

        gps = [gathers(b0 + r * _NB, r) for r in range(_RING)]
        for r in range(_RING):
            for cp in gps[r]:
                cp.wait()
            write(b0 + r * _NB, r)

        def body(g, carry):
            base = b0 + (_RING + g * _RING) * _NB
            gps = []
            for r in range(_RING):
                drain_write(r)
                gps.append(gathers(base + r * _NB, r))
            for r in range(_RING):
                for cp in gps[r]:
                    cp.wait()
                write(base + r * _NB, r)
            return carry

        lax.fori_loop(0, nrev, body, 0)

        base = b0 + (_RING + nrev * _RING) * _NB
        for r in range(ntail):
            drain_write(r)
            gp = gathers(base + r * _NB, r)
            for cp in gp:
                cp.wait()
            write(base + r * _NB, r)
        for r in range(_RING):
            drain_write(r)

    return k(x, cmap)


# speedup vs baseline: 1.0001x; 1.0001x over previous
import functools

import jax
import jax.numpy as jnp
from jax import lax
from jax.experimental import pallas as pl
from jax.experimental.pallas import tpu as pltpu
from jax.experimental.pallas import tpu_sc as plsc

_B, _C, _T = 16384, 14, 200
_NB = 8
_RING = 3


def kernel(chromatogram_batch, perm):
    x = chromatogram_batch
    p = perm.astype(jnp.int32)
    cmap = jnp.concatenate([
        p,
        jnp.array([6], jnp.int32),
        p + 7,
        jnp.array([13], jnp.int32),
        jnp.array([0, 0], jnp.int32),
    ])

    info = plsc.get_sparse_core_info()
    nc, ns = info.num_cores, info.num_subcores
    nw = nc * ns
    bw = _B // nw
    nchunks = bw // _NB
    nrev = (nchunks - _RING) // _RING
    ntail = nchunks - _RING - nrev * _RING
    mesh = plsc.VectorSubcoreMesh(core_axis_name="c", subcore_axis_name="s")

    @functools.partial(
        pl.kernel,
        mesh=mesh,
        out_type=jax.ShapeDtypeStruct((_B, _C, _T), jnp.float32),
        scratch_types=(
            [pltpu.VMEM((16,), jnp.int32)]
            + [pltpu.VMEM((_NB, _C, _T), jnp.float32) for _ in range(_RING)]
            + [pltpu.SemaphoreType.DMA for _ in range(2 * _RING)]
        ),
    )
    def k(x_hbm, cmap_hbm, out_hbm, cmap_v, *bufs_and_sems):
        bufs = bufs_and_sems[:_RING]
        gsems = bufs_and_sems[_RING:2 * _RING]
        wsems = bufs_and_sems[2 * _RING:]
        wid = lax.axis_index("s") * nc + lax.axis_index("c")
        b0 = wid * bw
        pltpu.sync_copy(cmap_hbm, cmap_v)
        cmapv = cmap_v[...]
        srcs = [cmapv[c] for c in range(_C)]

        def gathers(base, r):
            return [
                pltpu.async_copy(
                    x_hbm.at[pl.ds(base, _NB), pl.ds(srcs[c], 1)],
                    bufs[r].at[pl.ds(0, _NB), pl.ds(c, 1)],
                    gsems[r],
                )
                for c in range(_C)
            ]

        def write(base, r):
            pltpu.async_copy(bufs[r], out_hbm.at[pl.ds(base, _NB)], wsems[r])

        def drain_write(r):
            pltpu.make_async_copy(
                x_hbm.at[pl.ds(0, _NB)], bufs[r], wsems[r]
            ).wait()

        gps = [gathers(b0 + r * _NB, r) for r in range(_RING)]
        for r in range(_RING):
            for cp in gps[r]:
                cp.wait()
            write(b0 + r * _NB, r)

        def body(g, carry):
            base = b0 + (_RING + g * _RING) * _NB
            gps = []
            for r in range(_RING):
                drain_write(r)
                gps.append(gathers(base + r * _NB, r))
            for r in range(_RING):
                for cp in gps[r]:
                    cp.wait()
                write(base + r * _NB, r)
            return carry

        lax.fori_loop(0, nrev, body, 0)

        base = b0 + (_RING + nrev * _RING) * _NB
        for r in range(ntail):
            drain_write(r)
            gp = gathers(base + r * _NB, r)
            for cp in gp:
                cp.wait()
            write(base + r * _NB, r)
        for r in range(_RING):
            drain_write(r)

    return k(x, cmap)
